# Initial kernel scaffold; baseline (speedup 1.0000x reference)
import jax
import jax.numpy as jnp
from jax.experimental import pallas as pl


def kernel(token_ids, weights):
    raise NotImplementedError("write your pallas kernel here")


# speedup vs baseline: 1.7982x; 1.7982x over previous
import functools

import jax
import jax.numpy as jnp
from jax import lax
from jax.experimental import pallas as pl
from jax.experimental.pallas import tpu as pltpu
from jax.experimental.pallas import tpu_sc as plsc

_NC = 2
_NS = 16
_NW = _NC * _NS

_BATCH = 16384
_HIST = 50
_D = 64
_B = _BATCH * _HIST
_B_PER_W = _B // _NW
_CHUNK = 512
_N_CHUNKS = _B_PER_W // _CHUNK


def _gather_body(idx_hbm, table_hbm, out_hbm, idx_v, rows_v, sem):
    wid = lax.axis_index("s") * _NC + lax.axis_index("c")
    base = wid * _B_PER_W

    @pl.loop(0, _N_CHUNKS)
    def _chunk(g):
        off = base + g * _CHUNK
        pltpu.sync_copy(idx_hbm.at[pl.ds(off, _CHUNK)], idx_v)
        pltpu.async_copy(table_hbm.at[idx_v], rows_v, sem).wait()
        pltpu.sync_copy(rows_v, out_hbm.at[pl.ds(off, _CHUNK)])


_gather = functools.partial(
    pl.kernel,
    out_type=jax.ShapeDtypeStruct((_B, _D), jnp.float32),
    mesh=plsc.VectorSubcoreMesh(core_axis_name="c", subcore_axis_name="s"),
    scratch_types=[
        pltpu.VMEM((_CHUNK,), jnp.int32),
        pltpu.VMEM((_CHUNK, _D), jnp.float32),
        pltpu.SemaphoreType.DMA,
    ],
    compiler_params=pltpu.CompilerParams(use_tc_tiling_on_sc=False),
)(_gather_body)


def kernel(token_ids, weights):
    flat_ids = jnp.reshape(token_ids, (_B,)).astype(jnp.int32)
    out = _gather(flat_ids, weights)
    return jnp.reshape(out, (_BATCH, _HIST, _D))


# speedup vs baseline: 1.8778x; 1.0442x over previous
import jax
import jax.numpy as jnp
from jax import lax
from jax.experimental import pallas as pl
from jax.experimental.pallas import tpu as pltpu
from jax.experimental.pallas import tpu_sc as plsc

_NC = 2
_NS = 16
_NW = _NC * _NS

_BATCH = 16384
_HIST = 50
_D = 64
_B = _BATCH * _HIST
_B_PER_W = _B // _NW
_CHUNK = 256
_N_CHUNKS = _B_PER_W // _CHUNK
_NBUF = 4


def _gather_body(idx_hbm, table_hbm, out_hbm, idx_v, rows_v,
                 sg0, sg1, sg2, sg3, sw0, sw1, sw2, sw3):
    sg = (sg0, sg1, sg2, sg3)
    sw = (sw0, sw1, sw2, sw3)
    wid = lax.axis_index("s") * _NC + lax.axis_index("c")
    base = wid * _B_PER_W

    pltpu.sync_copy(idx_hbm.at[pl.ds(base, _B_PER_W)], idx_v)

    def g_desc(k, b):
        return pltpu.make_async_copy(
            table_hbm.at[idx_v.at[pl.ds(k * _CHUNK, _CHUNK)]],
            rows_v.at[b], sg[b])

    def w_desc(k, b):
        return pltpu.make_async_copy(
            rows_v.at[b],
            out_hbm.at[pl.ds(base + k * _CHUNK, _CHUNK)], sw[b])

    g_desc(0, 0).start()
    g_desc(1, 1).start()
    g_desc(2, 2).start()
    g_desc(3, 3).start()
    g_desc(0, 0).wait()
    w_desc(0, 0).start()

    @pl.loop(0, (_N_CHUNKS - 4) // 4)
    def _grp(i):
        for b in range(4):
            k = 4 * i + 1 + b
            w_desc(k - 1, b).wait()
            g_desc(k + 3, b).start()
            g_desc(k, (b + 1) % 4).wait()
            w_desc(k, (b + 1) % 4).start()

    kk = _N_CHUNKS - 3
    w_desc(kk - 1, 0).wait()
    for k, b in ((kk, 1), (kk + 1, 2), (kk + 2, 3)):
        g_desc(k, b).wait()
        w_desc(k, b).start()
    for k, b in ((kk, 1), (kk + 1, 2), (kk + 2, 3)):
        w_desc(k, b).wait()


_gather = pl.kernel(
    _gather_body,
    out_type=jax.ShapeDtypeStruct((_B, _D), jnp.float32),
    mesh=plsc.VectorSubcoreMesh(core_axis_name="c", subcore_axis_name="s"),
    scratch_types=[
        pltpu.VMEM((_B_PER_W,), jnp.int32),
        pltpu.VMEM((_NBUF, _CHUNK, _D), jnp.float32),
    ] + [pltpu.SemaphoreType.DMA] * (2 * _NBUF),
    compiler_params=pltpu.CompilerParams(use_tc_tiling_on_sc=False),
)


def kernel(token_ids, weights):
    flat_ids = jnp.reshape(token_ids, (_B,)).astype(jnp.int32)
    out = _gather(flat_ids, weights)
    return jnp.reshape(out, (_BATCH, _HIST, _D))


# speedup vs baseline: 1.9648x; 1.0463x over previous
import jax
import jax.numpy as jnp
from jax import lax
from jax.experimental import pallas as pl
from jax.experimental.pallas import tpu as pltpu
from jax.experimental.pallas import tpu_sc as plsc

_NC = 2
_NS = 16
_NW = _NC * _NS

_BATCH = 16384
_HIST = 50
_D = 64
_BP = _BATCH // _NW
_CHUNK = 256
_CPH = _BP // _CHUNK
_N_CHUNKS = _HIST * _CPH
_NBUF = 4


def _gather_body(idx_hbm, table_hbm, out_hbm, idx_v, rows_v,
                 sg0, sg1, sg2, sg3, sw0, sw1, sw2, sw3):
    sg = (sg0, sg1, sg2, sg3)
    sw = (sw0, sw1, sw2, sw3)
    wid = lax.axis_index("s") * _NC + lax.axis_index("c")
    bbase = wid * _BP

    pltpu.sync_copy(idx_hbm.at[:, pl.ds(bbase, _BP)], idx_v)

    def g_desc(k, b):
        h = k // _CPH
        boff = (k % _CPH) * _CHUNK
        return pltpu.make_async_copy(
            table_hbm.at[idx_v.at[h, pl.ds(boff, _CHUNK)]],
            rows_v.at[b], sg[b])

    def w_desc(k, b):
        h = k // _CPH
        boff = (k % _CPH) * _CHUNK
        return pltpu.make_async_copy(
            rows_v.at[b],
            out_hbm.at[h, pl.ds(bbase + boff, _CHUNK)], sw[b])

    g_desc(0, 0).start()
    g_desc(1, 1).start()
    g_desc(2, 2).start()
    g_desc(3, 3).start()
    g_desc(0, 0).wait()
    w_desc(0, 0).start()

    @pl.loop(0, (_N_CHUNKS - 4) // 4)
    def _grp(i):
        for b in range(4):
            k = 4 * i + 1 + b
            w_desc(k - 1, b).wait()
            g_desc(k + 3, b).start()
            g_desc(k, (b + 1) % 4).wait()
            w_desc(k, (b + 1) % 4).start()

    kk = _N_CHUNKS - 3
    w_desc(kk - 1, 0).wait()
    for k, b in ((kk, 1), (kk + 1, 2), (kk + 2, 3)):
        g_desc(k, b).wait()
        w_desc(k, b).start()
    for k, b in ((kk, 1), (kk + 1, 2), (kk + 2, 3)):
        w_desc(k, b).wait()


_gather = pl.kernel(
    _gather_body,
    out_type=jax.ShapeDtypeStruct((_HIST, _BATCH, _D), jnp.float32),
    mesh=plsc.VectorSubcoreMesh(core_axis_name="c", subcore_axis_name="s"),
    scratch_types=[
        pltpu.VMEM((_HIST, _BP), jnp.int32),
        pltpu.VMEM((_NBUF, _CHUNK, _D), jnp.float32),
    ] + [pltpu.SemaphoreType.DMA] * (2 * _NBUF),
    compiler_params=pltpu.CompilerParams(use_tc_tiling_on_sc=False),
)


def kernel(token_ids, weights):
    ids_hm = jnp.swapaxes(token_ids, 0, 1)
    out_hm = _gather(ids_hm, weights)
    return jnp.transpose(out_hm, (1, 0, 2))
